# Initial kernel scaffold; baseline (speedup 1.0000x reference)
import functools

import jax
import jax.numpy as jnp
from jax import lax
from jax.experimental import pallas as pl
from jax.experimental.pallas import tpu as pltpu
from jax.experimental.pallas import tpu_sc as plsc

N = 10000
E = 160000
D_IN = 128
D = 512
A = 7
FC = 512
B = 1024

DC = 128
NCH = D // DC
NC = 2
NS = 16
NW = NC * NS
EBLK = 128
NBLK = 40
EPT = EBLK * NBLK
EPAD = EPT * NW
NACC = 10016
ZROWS = 313
NPT = N // NS

BLKR = 1000
NGRID = N // BLKR

_mesh = plsc.VectorSubcoreMesh(core_axis_name="c", subcore_axis_name="s")


def _sc_deg_body(dst_hbm, zeros8_hbm, ones8_hbm, out_hbm,
                 dst_v, zer_v, one_v, acc, sem):
    cid = lax.axis_index("c")
    sid = lax.axis_index("s")
    wid = cid * NS + sid

    pltpu.sync_copy(dst_hbm.at[wid], dst_v)
    pltpu.sync_copy(zeros8_hbm, zer_v)
    pltpu.sync_copy(ones8_hbm, one_v)
    pltpu.sync_copy(zer_v, acc.at[pl.ds(sid * 626, 626)])
    plsc.subcore_barrier()

    @pl.loop(0, NBLK)
    def _edge_block(j):
        pltpu.sync_copy(one_v, acc.at[dst_v.at[j]], add=True)

    plsc.subcore_barrier()
    pltpu.sync_copy(acc.at[pl.ds(sid * NPT, NPT)],
                    out_hbm.at[cid, pl.ds(sid * NPT, NPT)])


_sc_deg = pl.kernel(
    _sc_deg_body,
    out_type=jax.ShapeDtypeStruct((NC, N, 8), jnp.float32),
    mesh=_mesh,
    scratch_types=[
        pltpu.VMEM((NBLK, EBLK), jnp.int32),
        pltpu.VMEM((626, 8), jnp.float32),
        pltpu.VMEM((EBLK, 8), jnp.float32),
        pltpu.VMEM_SHARED((NACC, 8), jnp.float32),
        pltpu.SemaphoreType.DMA,
    ],
)


def _sc_agg_body(hp0, hp1, hp2, hp3, src_hbm, dst_hbm, zeros_hbm,
                 out0, out1, out2, out3,
                 src_v, dst_v, rows_v, zer_v, acc, sem):
    cid = lax.axis_index("c")
    sid = lax.axis_index("s")
    wid = cid * NS + sid

    pltpu.sync_copy(src_hbm.at[wid], src_v)
    pltpu.sync_copy(dst_hbm.at[wid], dst_v)
    pltpu.sync_copy(zeros_hbm, zer_v)

    for hp_c, out_c in ((hp0, out0), (hp1, out1), (hp2, out2), (hp3, out3)):
        pltpu.sync_copy(zer_v, acc.at[pl.ds(sid * 626, ZROWS)])
        pltpu.sync_copy(zer_v, acc.at[pl.ds(sid * 626 + ZROWS, ZROWS)])
        plsc.subcore_barrier()

        @pl.loop(0, NBLK)
        def _edge_block(j):
            pltpu.async_copy(hp_c.at[src_v.at[j]], rows_v, sem).wait()
            pltpu.sync_copy(rows_v, acc.at[dst_v.at[j]], add=True)

        plsc.subcore_barrier()
        pltpu.sync_copy(acc.at[pl.ds(sid * NPT, NPT)],
                        out_c.at[cid, pl.ds(sid * NPT, NPT)])
        plsc.subcore_barrier()


_sc_agg = pl.kernel(
    _sc_agg_body,
    out_type=[jax.ShapeDtypeStruct((NC, N, DC), jnp.float32)
              for _ in range(NCH)],
    mesh=_mesh,
    scratch_types=[
        pltpu.VMEM((NBLK, EBLK), jnp.int32),
        pltpu.VMEM((NBLK, EBLK), jnp.int32),
        pltpu.VMEM((EBLK, DC), jnp.float32),
        pltpu.VMEM((ZROWS, DC), jnp.float32),
        pltpu.VMEM_SHARED((NACC, DC), jnp.float32),
        pltpu.SemaphoreType.DMA,
    ],
)


def _leaky(x):
    return jnp.maximum(x, 0.01 * x)


def _ln(x, g, b):
    m = jnp.mean(x, axis=-1, keepdims=True)
    v = jnp.mean((x - m) * (x - m), axis=-1, keepdims=True)
    return (x - m) * lax.rsqrt(v + 1e-5) * g + b


def _tc_pre_body(x_ref, w_ref, deg_ref,
                 hp0_ref, hp1_ref, hp2_ref, hp3_ref, dinv_ref):
    deg = deg_ref[0] + deg_ref[1] + 1.0
    dinv8 = lax.rsqrt(deg)
    dinv = dinv8[:, 0:1]
    h = jnp.dot(x_ref[...], w_ref[...], preferred_element_type=jnp.float32)
    hp = h * dinv
    for c, ref in enumerate((hp0_ref, hp1_ref, hp2_ref, hp3_ref)):
        ref[...] = hp[:, c * DC:(c + 1) * DC]
    dinv_ref[...] = dinv8


_tc_pre = pl.pallas_call(
    _tc_pre_body,
    grid=(NGRID,),
    in_specs=[
        pl.BlockSpec((BLKR, D_IN), lambda i: (i, 0)),
        pl.BlockSpec((D_IN, D), lambda i: (0, 0)),
        pl.BlockSpec((NC, BLKR, 8), lambda i: (0, i, 0)),
    ],
    out_specs=[pl.BlockSpec((BLKR, DC), lambda i: (i, 0))] * NCH
    + [pl.BlockSpec((BLKR, 8), lambda i: (i, 0))],
    out_shape=[jax.ShapeDtypeStruct((N, DC), jnp.float32)
               for _ in range(NCH)]
    + [jax.ShapeDtypeStruct((N, 8), jnp.float32)],
)


def _tc_mid_body(a0, a1, a2, a3, h0, h1, h2, h3, dinv_ref, b_ref, w_ref,
                 o0, o1, o2, o3, *, act):
    dinv = dinv_ref[:, 0:1]
    parts = []
    for c, (a_c, h_c) in enumerate(((a0, h0), (a1, h1), (a2, h2), (a3, h3))):
        s = a_c[0] + a_c[1] + h_c[...]
        pre = s * dinv + b_ref[0:1, c * DC:(c + 1) * DC]
        parts.append(_leaky(pre) if act else pre)
    g = jnp.concatenate(parts, axis=1)
    h = jnp.dot(g, w_ref[...], preferred_element_type=jnp.float32)
    hp = h * dinv
    for c, ref in enumerate((o0, o1, o2, o3)):
        ref[...] = hp[:, c * DC:(c + 1) * DC]


def _make_tc_mid(act):
    return pl.pallas_call(
        functools.partial(_tc_mid_body, act=act),
        grid=(NGRID,),
        in_specs=[pl.BlockSpec((NC, BLKR, DC), lambda i: (0, i, 0))] * NCH
        + [pl.BlockSpec((BLKR, DC), lambda i: (i, 0))] * NCH
        + [
            pl.BlockSpec((BLKR, 8), lambda i: (i, 0)),
            pl.BlockSpec((1, D), lambda i: (0, 0)),
            pl.BlockSpec((D, D), lambda i: (0, 0)),
        ],
        out_specs=[pl.BlockSpec((BLKR, DC), lambda i: (i, 0))] * NCH,
        out_shape=[jax.ShapeDtypeStruct((N, DC), jnp.float32)
                   for _ in range(NCH)],
    )


_tc_mid_l0 = _make_tc_mid(False)
_tc_mid_act = _make_tc_mid(True)


def _tc_final_body(a0, a1, a2, a3, h0, h1, h2, h3, dinv_ref, b_ref,
                   act_ref, ln2g, ln2b, afc1w, afc1b, ln5g, ln5b,
                   afc2w, afc2b, afc3w, afc3b, fc1w, fc1b, ln4g, ln4b,
                   fc2w, fc2b, out_ref, mx_ref):
    i = pl.program_id(0)
    dinv = dinv_ref[:, 0:1]
    parts = []
    for c, (a_c, h_c) in enumerate(((a0, h0), (a1, h1), (a2, h2), (a3, h3))):
        s = a_c[0] + a_c[1] + h_c[...]
        pre = s * dinv + b_ref[0:1, c * DC:(c + 1) * DC]
        parts.append(_leaky(pre))
    g = jnp.concatenate(parts, axis=1)
    gmax = jnp.max(g, axis=0, keepdims=True)

    @pl.when(i == 0)
    def _():
        mx_ref[...] = gmax

    @pl.when(i > 0)
    def _():
        mx_ref[...] = jnp.maximum(mx_ref[...], gmax)

    @pl.when(i == NGRID - 1)
    def _():
        v = _ln(mx_ref[...], ln2g[...], ln2b[...])
        a = _leaky(jnp.dot(act_ref[...], afc1w[...],
                           preferred_element_type=jnp.float32) + afc1b[...])
        a = _ln(a, ln5g[...], ln5b[...])
        a = _leaky(jnp.dot(a, afc2w[...],
                           preferred_element_type=jnp.float32) + afc2b[...])
        a = jnp.dot(a, afc3w[...],
                    preferred_element_type=jnp.float32) + afc3b[...]
        z = v * a
        z = _leaky(jnp.dot(z, fc1w[...],
                           preferred_element_type=jnp.float32) + fc1b[...])
        z = _ln(z, ln4g[...], ln4b[...])
        out_ref[...] = jnp.dot(z, fc2w[...],
                               preferred_element_type=jnp.float32) + fc2b[...]


def _full(shape):
    return pl.BlockSpec(shape, lambda i: tuple(0 for _ in shape))


_tc_final = pl.pallas_call(
    _tc_final_body,
    grid=(NGRID,),
    in_specs=[pl.BlockSpec((NC, BLKR, DC), lambda i: (0, i, 0))] * NCH
    + [pl.BlockSpec((BLKR, DC), lambda i: (i, 0))] * NCH
    + [
        pl.BlockSpec((BLKR, 8), lambda i: (i, 0)),
        _full((1, D)),
        _full((B, A)),
        _full((1, D)), _full((1, D)),
        _full((A, 8 * A)), _full((1, 8 * A)),
        _full((1, 8 * A)), _full((1, 8 * A)),
        _full((8 * A, 8 * A)), _full((1, 8 * A)),
        _full((8 * A, D)), _full((1, D)),
        _full((D, FC)), _full((1, FC)),
        _full((1, FC)), _full((1, FC)),
        _full((FC, 1)), _full((1, 1)),
    ],
    out_specs=pl.BlockSpec((B, 1), lambda i: (0, 0)),
    out_shape=jax.ShapeDtypeStruct((B, 1), jnp.float32),
    scratch_shapes=[pltpu.VMEM((1, D), jnp.float32)],
)


def kernel(x, action, edge_index, W0, b0, W1, b1, W2, b2, W3, b3,
           ln2_g, ln2_b, afc1_W, afc1_b, ln5_g, ln5_b, afc2_W, afc2_b,
           afc3_W, afc3_b, fc1_W, fc1_b, ln4_g, ln4_b, fc2_W, fc2_b):
    src = edge_index[0]
    dst = edge_index[1]
    pad = EPAD - E
    src3 = jnp.concatenate(
        [src, jnp.zeros((pad,), jnp.int32)]).reshape(NW, NBLK, EBLK)
    dst3 = jnp.concatenate(
        [dst, jnp.full((pad,), N, jnp.int32)]).reshape(NW, NBLK, EBLK)

    zeros8 = jnp.zeros((626, 8), jnp.float32)
    ones8 = jnp.ones((EBLK, 8), jnp.float32)
    zeros128 = jnp.zeros((ZROWS, DC), jnp.float32)

    r1 = lambda a: a.reshape(1, -1)

    deg = _sc_deg(dst3, zeros8, ones8)
    *hp, dinv = _tc_pre(x, W0, deg)

    agg = _sc_agg(*hp, src3, dst3, zeros128)
    hp = _tc_mid_l0(*agg, *hp, dinv, r1(b0), W1)

    agg = _sc_agg(*hp, src3, dst3, zeros128)
    hp = _tc_mid_act(*agg, *hp, dinv, r1(b1), W2)

    agg = _sc_agg(*hp, src3, dst3, zeros128)
    hp = _tc_mid_act(*agg, *hp, dinv, r1(b2), W3)

    agg = _sc_agg(*hp, src3, dst3, zeros128)
    out = _tc_final(*agg, *hp, dinv, r1(b3), action,
                    r1(ln2_g), r1(ln2_b), afc1_W, r1(afc1_b),
                    r1(ln5_g), r1(ln5_b), afc2_W, r1(afc2_b),
                    afc3_W, r1(afc3_b), fc1_W, r1(fc1_b),
                    r1(ln4_g), r1(ln4_b), fc2_W, r1(fc2_b))
    return out[:, 0]


# speedup vs baseline: 3.5109x; 3.5109x over previous
import functools

import jax
import jax.numpy as jnp
from jax import lax
from jax.experimental import pallas as pl
from jax.experimental.pallas import tpu as pltpu
from jax.experimental.pallas import tpu_sc as plsc

N = 10000
E = 160000
D_IN = 128
D = 512
A = 7
FC = 512
B = 1024

DC = 128
NCH = D // DC
NC = 2
NS = 16
NW = NC * NS
EBLK = 128
NBLK = 40
EPT = EBLK * NBLK
EPAD = EPT * NW
NACC = 10112
ZROWS = 632
ZBUF = 128
NPT = 624
NPT_LAST = N - NPT * (NS - 1)

BLKR = 1000
NGRID = N // BLKR

def _zero_stripe(acc, zer_v, sid):
    @pl.loop(0, 4)
    def _zero_chunk(k):
        off = pl.multiple_of(sid * ZROWS + k * ZBUF, 8)
        pltpu.sync_copy(zer_v, acc.at[pl.ds(off, ZBUF)])

    pltpu.sync_copy(
        zer_v.at[pl.ds(0, ZROWS - 4 * ZBUF)],
        acc.at[pl.ds(pl.multiple_of(sid * ZROWS + 4 * ZBUF, 8),
                     ZROWS - 4 * ZBUF)])


def _sc_deg_body(dst_hbm, zeros_hbm, ones_hbm, out_hbm,
                 dst_v, zer_v, one_v, acc, sem):
    cid = lax.axis_index("c")
    sid = lax.axis_index("s")
    wid = cid * NS + sid

    pltpu.sync_copy(dst_hbm.at[pl.ds(wid * NBLK, NBLK)], dst_v)
    pltpu.sync_copy(zeros_hbm, zer_v)
    pltpu.sync_copy(ones_hbm, one_v)

    _zero_stripe(acc, zer_v, sid)
    plsc.subcore_barrier()

    @pl.loop(0, NBLK)
    def _blk(j):
        pltpu.sync_copy(one_v, acc.at[dst_v.at[j]], add=True)

    plsc.subcore_barrier()
    s = pl.multiple_of(sid * ZROWS, 8)
    pltpu.sync_copy(acc.at[pl.ds(s, ZROWS)],
                    out_hbm.at[pl.ds(cid * NACC + s, ZROWS)])


def _copy_out(acc, out_hbm, cid, sid):
    @pl.when(sid < NS - 1)
    def _():
        s = pl.multiple_of(sid * NPT, 8)
        pltpu.sync_copy(acc.at[pl.ds(s, NPT)],
                        out_hbm.at[pl.ds(cid * N + s, NPT)])

    @pl.when(sid == NS - 1)
    def _():
        s0 = NPT * (NS - 1)
        pltpu.sync_copy(acc.at[pl.ds(s0, NPT_LAST)],
                        out_hbm.at[pl.ds(cid * N + s0, NPT_LAST)])


@functools.cache
def _get_sc_deg():
    return pl.kernel(
        _sc_deg_body,
        out_type=jax.ShapeDtypeStruct((NC * NACC, DC), jnp.float32),
        mesh=plsc.VectorSubcoreMesh(core_axis_name="c", subcore_axis_name="s"),
        scratch_types=[
            pltpu.VMEM((NBLK, EBLK), jnp.int32),
            pltpu.VMEM((ZBUF, DC), jnp.float32),
            pltpu.VMEM((EBLK, DC), jnp.float32),
            pltpu.VMEM_SHARED((NACC, DC), jnp.float32),
            pltpu.SemaphoreType.DMA,
        ],
    )


def _sc_deg(*args):
    return _get_sc_deg()(*args)


def _sc_agg_body(hp0, hp1, hp2, hp3, src_hbm, dst_hbm, zeros_hbm,
                 out0, out1, out2, out3,
                 src_v, dst_v, rows_v, zer_v, acc, sem):
    cid = lax.axis_index("c")
    sid = lax.axis_index("s")
    wid = cid * NS + sid

    pltpu.sync_copy(src_hbm.at[pl.ds(wid * NBLK, NBLK)], src_v)
    pltpu.sync_copy(dst_hbm.at[pl.ds(wid * NBLK, NBLK)], dst_v)
    pltpu.sync_copy(zeros_hbm, zer_v)

    for hp_c, out_c in ((hp0, out0), (hp1, out1), (hp2, out2), (hp3, out3)):
        _zero_stripe(acc, zer_v, sid)
        plsc.subcore_barrier()

        @pl.loop(0, NBLK)
        def _edge_block(j):
            pltpu.async_copy(hp_c.at[src_v.at[j]], rows_v, sem).wait()
            pltpu.sync_copy(rows_v, acc.at[dst_v.at[j]], add=True)

        plsc.subcore_barrier()
        _copy_out(acc, out_c, cid, sid)
        plsc.subcore_barrier()


@functools.cache
def _get_sc_agg():
    return pl.kernel(
        _sc_agg_body,
        out_type=[jax.ShapeDtypeStruct((NC * N, DC), jnp.float32)
                  for _ in range(NCH)],
        mesh=plsc.VectorSubcoreMesh(core_axis_name="c", subcore_axis_name="s"),
        scratch_types=[
            pltpu.VMEM((NBLK, EBLK), jnp.int32),
            pltpu.VMEM((NBLK, EBLK), jnp.int32),
            pltpu.VMEM((EBLK, DC), jnp.float32),
            pltpu.VMEM((ZBUF, DC), jnp.float32),
            pltpu.VMEM_SHARED((NACC, DC), jnp.float32),
            pltpu.SemaphoreType.DMA,
        ],
    )


def _sc_agg(*args):
    return _get_sc_agg()(*args)


def _leaky(x):
    return jnp.maximum(x, 0.01 * x)


def _ln(x, g, b):
    m = jnp.mean(x, axis=-1, keepdims=True)
    v = jnp.mean((x - m) * (x - m), axis=-1, keepdims=True)
    return (x - m) * lax.rsqrt(v + 1e-5) * g + b


def _tc_pre_body(x_ref, w_ref, deg_ref,
                 hp0_ref, hp1_ref, hp2_ref, hp3_ref, dinv_ref):
    deg = deg_ref[0] + deg_ref[1] + 1.0
    dinv8 = lax.rsqrt(deg)[:, 0:8]
    dinv = dinv8[:, 0:1]
    h = jnp.dot(x_ref[...], w_ref[...], preferred_element_type=jnp.float32)
    hp = h * dinv
    for c, ref in enumerate((hp0_ref, hp1_ref, hp2_ref, hp3_ref)):
        ref[...] = hp[:, c * DC:(c + 1) * DC]
    dinv_ref[...] = dinv8


_tc_pre = pl.pallas_call(
    _tc_pre_body,
    grid=(NGRID,),
    in_specs=[
        pl.BlockSpec((BLKR, D_IN), lambda i: (i, 0)),
        pl.BlockSpec((D_IN, D), lambda i: (0, 0)),
        pl.BlockSpec((NC, BLKR, DC), lambda i: (0, i, 0)),
    ],
    out_specs=[pl.BlockSpec((BLKR, DC), lambda i: (i, 0))] * NCH
    + [pl.BlockSpec((BLKR, 8), lambda i: (i, 0))],
    out_shape=[jax.ShapeDtypeStruct((N, DC), jnp.float32)
               for _ in range(NCH)]
    + [jax.ShapeDtypeStruct((N, 8), jnp.float32)],
)


def _tc_mid_body(a0, a1, a2, a3, h0, h1, h2, h3, dinv_ref, b_ref, w_ref,
                 o0, o1, o2, o3, *, act):
    dinv = dinv_ref[:, 0:1]
    parts = []
    for c, (a_c, h_c) in enumerate(((a0, h0), (a1, h1), (a2, h2), (a3, h3))):
        s = a_c[0] + a_c[1] + h_c[...]
        pre = s * dinv + b_ref[0:1, c * DC:(c + 1) * DC]
        parts.append(_leaky(pre) if act else pre)
    g = jnp.concatenate(parts, axis=1)
    h = jnp.dot(g, w_ref[...], preferred_element_type=jnp.float32)
    hp = h * dinv
    for c, ref in enumerate((o0, o1, o2, o3)):
        ref[...] = hp[:, c * DC:(c + 1) * DC]


def _make_tc_mid(act):
    return pl.pallas_call(
        functools.partial(_tc_mid_body, act=act),
        grid=(NGRID,),
        in_specs=[pl.BlockSpec((NC, BLKR, DC), lambda i: (0, i, 0))] * NCH
        + [pl.BlockSpec((BLKR, DC), lambda i: (i, 0))] * NCH
        + [
            pl.BlockSpec((BLKR, 8), lambda i: (i, 0)),
            pl.BlockSpec((1, D), lambda i: (0, 0)),
            pl.BlockSpec((D, D), lambda i: (0, 0)),
        ],
        out_specs=[pl.BlockSpec((BLKR, DC), lambda i: (i, 0))] * NCH,
        out_shape=[jax.ShapeDtypeStruct((N, DC), jnp.float32)
                   for _ in range(NCH)],
    )


_tc_mid_l0 = _make_tc_mid(False)
_tc_mid_act = _make_tc_mid(True)


def _tc_final_body(a0, a1, a2, a3, h0, h1, h2, h3, dinv_ref, b_ref,
                   act_ref, ln2g, ln2b, afc1w, afc1b, ln5g, ln5b,
                   afc2w, afc2b, afc3w, afc3b, fc1w, fc1b, ln4g, ln4b,
                   fc2w, fc2b, out_ref, mx_ref):
    i = pl.program_id(0)
    dinv = dinv_ref[:, 0:1]
    parts = []
    for c, (a_c, h_c) in enumerate(((a0, h0), (a1, h1), (a2, h2), (a3, h3))):
        s = a_c[0] + a_c[1] + h_c[...]
        pre = s * dinv + b_ref[0:1, c * DC:(c + 1) * DC]
        parts.append(_leaky(pre))
    g = jnp.concatenate(parts, axis=1)
    gmax = jnp.max(g, axis=0, keepdims=True)

    @pl.when(i == 0)
    def _():
        mx_ref[...] = gmax

    @pl.when(i > 0)
    def _():
        mx_ref[...] = jnp.maximum(mx_ref[...], gmax)

    @pl.when(i == NGRID - 1)
    def _():
        v = _ln(mx_ref[...], ln2g[...], ln2b[...])
        a = _leaky(jnp.dot(act_ref[...], afc1w[...],
                           preferred_element_type=jnp.float32) + afc1b[...])
        a = _ln(a, ln5g[...], ln5b[...])
        a = _leaky(jnp.dot(a, afc2w[...],
                           preferred_element_type=jnp.float32) + afc2b[...])
        a = jnp.dot(a, afc3w[...],
                    preferred_element_type=jnp.float32) + afc3b[...]
        z = v * a
        z = _leaky(jnp.dot(z, fc1w[...],
                           preferred_element_type=jnp.float32) + fc1b[...])
        z = _ln(z, ln4g[...], ln4b[...])
        out_ref[...] = jnp.dot(z, fc2w[...],
                               preferred_element_type=jnp.float32) + fc2b[...]


def _full(shape):
    return pl.BlockSpec(shape, lambda i: tuple(0 for _ in shape))


_tc_final = pl.pallas_call(
    _tc_final_body,
    grid=(NGRID,),
    in_specs=[pl.BlockSpec((NC, BLKR, DC), lambda i: (0, i, 0))] * NCH
    + [pl.BlockSpec((BLKR, DC), lambda i: (i, 0))] * NCH
    + [
        pl.BlockSpec((BLKR, 8), lambda i: (i, 0)),
        _full((1, D)),
        _full((B, A)),
        _full((1, D)), _full((1, D)),
        _full((A, 8 * A)), _full((1, 8 * A)),
        _full((1, 8 * A)), _full((1, 8 * A)),
        _full((8 * A, 8 * A)), _full((1, 8 * A)),
        _full((8 * A, D)), _full((1, D)),
        _full((D, FC)), _full((1, FC)),
        _full((1, FC)), _full((1, FC)),
        _full((FC, 1)), _full((1, 1)),
    ],
    out_specs=pl.BlockSpec((B, 1), lambda i: (0, 0)),
    out_shape=jax.ShapeDtypeStruct((B, 1), jnp.float32),
    scratch_shapes=[pltpu.VMEM((1, D), jnp.float32)],
)


def kernel(x, action, edge_index, W0, b0, W1, b1, W2, b2, W3, b3,
           ln2_g, ln2_b, afc1_W, afc1_b, ln5_g, ln5_b, afc2_W, afc2_b,
           afc3_W, afc3_b, fc1_W, fc1_b, ln4_g, ln4_b, fc2_W, fc2_b):
    src = edge_index[0]
    dst = edge_index[1]
    pad = EPAD - E
    src3 = jnp.concatenate(
        [src, jnp.zeros((pad,), jnp.int32)]).reshape(NW * NBLK, EBLK)
    dst3 = jnp.concatenate(
        [dst, jnp.full((pad,), N, jnp.int32)]).reshape(NW * NBLK, EBLK)

    zeros128 = jnp.zeros((ZBUF, DC), jnp.float32)
    ones128 = jnp.ones((EBLK, DC), jnp.float32)

    r1 = lambda a: a.reshape(1, -1)
    r3 = lambda outs: [a.reshape(NC, N, DC) for a in outs]

    deg = _sc_deg(dst3, zeros128, ones128).reshape(NC, NACC, DC)[:, :N]
    *hp, dinv = _tc_pre(x, W0, deg)

    agg = r3(_sc_agg(*hp, src3, dst3, zeros128))
    hp = _tc_mid_l0(*agg, *hp, dinv, r1(b0), W1)

    agg = r3(_sc_agg(*hp, src3, dst3, zeros128))
    hp = _tc_mid_act(*agg, *hp, dinv, r1(b1), W2)

    agg = r3(_sc_agg(*hp, src3, dst3, zeros128))
    hp = _tc_mid_act(*agg, *hp, dinv, r1(b2), W3)

    agg = r3(_sc_agg(*hp, src3, dst3, zeros128))
    out = _tc_final(*agg, *hp, dinv, r1(b3), action,
                    r1(ln2_g), r1(ln2_b), afc1_W, r1(afc1_b),
                    r1(ln5_g), r1(ln5_b), afc2_W, r1(afc2_b),
                    afc3_W, r1(afc3_b), fc1_W, r1(fc1_b),
                    r1(ln4_g), r1(ln4_b), fc2_W, r1(fc2_b))
    return out[:, 0]


# speedup vs baseline: 3.6049x; 1.0268x over previous
import functools

import jax
import jax.numpy as jnp
from jax import lax
from jax.experimental import pallas as pl
from jax.experimental.pallas import tpu as pltpu
from jax.experimental.pallas import tpu_sc as plsc

N = 10000
E = 160000
D_IN = 128
D = 512
A = 7
FC = 512
B = 1024

DC = 128
NCH = D // DC
NC = 2
NS = 16
NW = NC * NS
EBLK = 128
NBLK = 40
EPT = EBLK * NBLK
EPAD = EPT * NW
NACC = 10112
ZROWS = 632
ZBUF = 128
NPT = 624
NPT_LAST = N - NPT * (NS - 1)

BLKR = 1000
NGRID = N // BLKR

def _zero_stripe(acc, zer_v, sid):
    @pl.loop(0, 4)
    def _zero_chunk(k):
        off = pl.multiple_of(sid * ZROWS + k * ZBUF, 8)
        pltpu.sync_copy(zer_v, acc.at[pl.ds(off, ZBUF)])

    pltpu.sync_copy(
        zer_v.at[pl.ds(0, ZROWS - 4 * ZBUF)],
        acc.at[pl.ds(pl.multiple_of(sid * ZROWS + 4 * ZBUF, 8),
                     ZROWS - 4 * ZBUF)])


def _sc_deg_body(dst_hbm, zeros_hbm, ones_hbm, out_hbm,
                 dst_v, zer_v, one_v, acc, sem):
    cid = lax.axis_index("c")
    sid = lax.axis_index("s")
    wid = cid * NS + sid

    pltpu.sync_copy(dst_hbm.at[pl.ds(wid * NBLK, NBLK)], dst_v)
    pltpu.sync_copy(zeros_hbm, zer_v)
    pltpu.sync_copy(ones_hbm, one_v)

    _zero_stripe(acc, zer_v, sid)
    plsc.subcore_barrier()

    @pl.loop(0, NBLK)
    def _blk(j):
        pltpu.sync_copy(one_v, acc.at[dst_v.at[j]], add=True)

    plsc.subcore_barrier()
    s = pl.multiple_of(sid * ZROWS, 8)
    pltpu.sync_copy(acc.at[pl.ds(s, ZROWS)],
                    out_hbm.at[pl.ds(cid * NACC + s, ZROWS)])


def _copy_out(acc, out_hbm, cid, sid):
    @pl.when(sid < NS - 1)
    def _():
        s = pl.multiple_of(sid * NPT, 8)
        pltpu.sync_copy(acc.at[pl.ds(s, NPT)],
                        out_hbm.at[pl.ds(cid * N + s, NPT)])

    @pl.when(sid == NS - 1)
    def _():
        s0 = NPT * (NS - 1)
        pltpu.sync_copy(acc.at[pl.ds(s0, NPT_LAST)],
                        out_hbm.at[pl.ds(cid * N + s0, NPT_LAST)])


@functools.cache
def _get_sc_deg():
    return pl.kernel(
        _sc_deg_body,
        out_type=jax.ShapeDtypeStruct((NC * NACC, DC), jnp.float32),
        mesh=plsc.VectorSubcoreMesh(core_axis_name="c", subcore_axis_name="s"),
        scratch_types=[
            pltpu.VMEM((NBLK, EBLK), jnp.int32),
            pltpu.VMEM((ZBUF, DC), jnp.float32),
            pltpu.VMEM((EBLK, DC), jnp.float32),
            pltpu.VMEM_SHARED((NACC, DC), jnp.float32),
            pltpu.SemaphoreType.DMA,
        ],
    )


def _sc_deg(*args):
    return _get_sc_deg()(*args)


def _sc_agg_body(hp0, hp1, hp2, hp3, src_hbm, dst_hbm, zeros_hbm,
                 out0, out1, out2, out3,
                 src_v, dst_v, rows_a, rows_b, acc, sem_a, sem_b):
    cid = lax.axis_index("c")
    sid = lax.axis_index("s")
    wid = cid * NS + sid

    pltpu.sync_copy(src_hbm.at[pl.ds(wid * NBLK, NBLK)], src_v)
    pltpu.sync_copy(dst_hbm.at[pl.ds(wid * NBLK, NBLK)], dst_v)

    for hp_c, out_c in ((hp0, out0), (hp1, out1), (hp2, out2), (hp3, out3)):
        pltpu.sync_copy(zeros_hbm, rows_a)
        _zero_stripe(acc, rows_a, sid)
        plsc.subcore_barrier()

        @pl.loop(0, NBLK // 2)
        def _edge_pair(p):
            j0 = p * 2
            ca = pltpu.async_copy(hp_c.at[src_v.at[j0]], rows_a, sem_a)
            cb = pltpu.async_copy(hp_c.at[src_v.at[j0 + 1]], rows_b, sem_b)
            ca.wait()
            pltpu.sync_copy(rows_a, acc.at[dst_v.at[j0]], add=True)
            cb.wait()
            pltpu.sync_copy(rows_b, acc.at[dst_v.at[j0 + 1]], add=True)

        plsc.subcore_barrier()
        _copy_out(acc, out_c, cid, sid)
        plsc.subcore_barrier()


@functools.cache
def _get_sc_agg():
    return pl.kernel(
        _sc_agg_body,
        out_type=[jax.ShapeDtypeStruct((NC * N, DC), jnp.float32)
                  for _ in range(NCH)],
        mesh=plsc.VectorSubcoreMesh(core_axis_name="c", subcore_axis_name="s"),
        scratch_types=[
            pltpu.VMEM((NBLK, EBLK), jnp.int32),
            pltpu.VMEM((NBLK, EBLK), jnp.int32),
            pltpu.VMEM((EBLK, DC), jnp.float32),
            pltpu.VMEM((EBLK, DC), jnp.float32),
            pltpu.VMEM_SHARED((NACC, DC), jnp.float32),
            pltpu.SemaphoreType.DMA,
            pltpu.SemaphoreType.DMA,
        ],
    )


def _sc_agg(*args):
    return _get_sc_agg()(*args)


def _leaky(x):
    return jnp.maximum(x, 0.01 * x)


def _ln(x, g, b):
    m = jnp.mean(x, axis=-1, keepdims=True)
    v = jnp.mean((x - m) * (x - m), axis=-1, keepdims=True)
    return (x - m) * lax.rsqrt(v + 1e-5) * g + b


def _tc_pre_body(x_ref, w_ref, deg_ref,
                 hp0_ref, hp1_ref, hp2_ref, hp3_ref, dinv_ref):
    deg = deg_ref[0] + deg_ref[1] + 1.0
    dinv8 = lax.rsqrt(deg)[:, 0:8]
    dinv = dinv8[:, 0:1]
    h = jnp.dot(x_ref[...], w_ref[...], preferred_element_type=jnp.float32)
    hp = h * dinv
    for c, ref in enumerate((hp0_ref, hp1_ref, hp2_ref, hp3_ref)):
        ref[...] = hp[:, c * DC:(c + 1) * DC]
    dinv_ref[...] = dinv8


_tc_pre = pl.pallas_call(
    _tc_pre_body,
    grid=(NGRID,),
    in_specs=[
        pl.BlockSpec((BLKR, D_IN), lambda i: (i, 0)),
        pl.BlockSpec((D_IN, D), lambda i: (0, 0)),
        pl.BlockSpec((NC, BLKR, DC), lambda i: (0, i, 0)),
    ],
    out_specs=[pl.BlockSpec((BLKR, DC), lambda i: (i, 0))] * NCH
    + [pl.BlockSpec((BLKR, 8), lambda i: (i, 0))],
    out_shape=[jax.ShapeDtypeStruct((N, DC), jnp.float32)
               for _ in range(NCH)]
    + [jax.ShapeDtypeStruct((N, 8), jnp.float32)],
)


def _tc_mid_body(a0, a1, a2, a3, h0, h1, h2, h3, dinv_ref, b_ref, w_ref,
                 o0, o1, o2, o3, *, act):
    dinv = dinv_ref[:, 0:1]
    parts = []
    for c, (a_c, h_c) in enumerate(((a0, h0), (a1, h1), (a2, h2), (a3, h3))):
        s = a_c[0] + a_c[1] + h_c[...]
        pre = s * dinv + b_ref[0:1, c * DC:(c + 1) * DC]
        parts.append(_leaky(pre) if act else pre)
    g = jnp.concatenate(parts, axis=1)
    h = jnp.dot(g, w_ref[...], preferred_element_type=jnp.float32)
    hp = h * dinv
    for c, ref in enumerate((o0, o1, o2, o3)):
        ref[...] = hp[:, c * DC:(c + 1) * DC]


def _make_tc_mid(act):
    return pl.pallas_call(
        functools.partial(_tc_mid_body, act=act),
        grid=(NGRID,),
        in_specs=[pl.BlockSpec((NC, BLKR, DC), lambda i: (0, i, 0))] * NCH
        + [pl.BlockSpec((BLKR, DC), lambda i: (i, 0))] * NCH
        + [
            pl.BlockSpec((BLKR, 8), lambda i: (i, 0)),
            pl.BlockSpec((1, D), lambda i: (0, 0)),
            pl.BlockSpec((D, D), lambda i: (0, 0)),
        ],
        out_specs=[pl.BlockSpec((BLKR, DC), lambda i: (i, 0))] * NCH,
        out_shape=[jax.ShapeDtypeStruct((N, DC), jnp.float32)
                   for _ in range(NCH)],
    )


_tc_mid_l0 = _make_tc_mid(False)
_tc_mid_act = _make_tc_mid(True)


def _tc_final_body(a0, a1, a2, a3, h0, h1, h2, h3, dinv_ref, b_ref,
                   act_ref, ln2g, ln2b, afc1w, afc1b, ln5g, ln5b,
                   afc2w, afc2b, afc3w, afc3b, fc1w, fc1b, ln4g, ln4b,
                   fc2w, fc2b, out_ref, mx_ref):
    i = pl.program_id(0)
    dinv = dinv_ref[:, 0:1]
    parts = []
    for c, (a_c, h_c) in enumerate(((a0, h0), (a1, h1), (a2, h2), (a3, h3))):
        s = a_c[0] + a_c[1] + h_c[...]
        pre = s * dinv + b_ref[0:1, c * DC:(c + 1) * DC]
        parts.append(_leaky(pre))
    g = jnp.concatenate(parts, axis=1)
    gmax = jnp.max(g, axis=0, keepdims=True)

    @pl.when(i == 0)
    def _():
        mx_ref[...] = gmax

    @pl.when(i > 0)
    def _():
        mx_ref[...] = jnp.maximum(mx_ref[...], gmax)

    @pl.when(i == NGRID - 1)
    def _():
        v = _ln(mx_ref[...], ln2g[...], ln2b[...])
        a = _leaky(jnp.dot(act_ref[...], afc1w[...],
                           preferred_element_type=jnp.float32) + afc1b[...])
        a = _ln(a, ln5g[...], ln5b[...])
        a = _leaky(jnp.dot(a, afc2w[...],
                           preferred_element_type=jnp.float32) + afc2b[...])
        a = jnp.dot(a, afc3w[...],
                    preferred_element_type=jnp.float32) + afc3b[...]
        z = v * a
        z = _leaky(jnp.dot(z, fc1w[...],
                           preferred_element_type=jnp.float32) + fc1b[...])
        z = _ln(z, ln4g[...], ln4b[...])
        out_ref[...] = jnp.dot(z, fc2w[...],
                               preferred_element_type=jnp.float32) + fc2b[...]


def _full(shape):
    return pl.BlockSpec(shape, lambda i: tuple(0 for _ in shape))


_tc_final = pl.pallas_call(
    _tc_final_body,
    grid=(NGRID,),
    in_specs=[pl.BlockSpec((NC, BLKR, DC), lambda i: (0, i, 0))] * NCH
    + [pl.BlockSpec((BLKR, DC), lambda i: (i, 0))] * NCH
    + [
        pl.BlockSpec((BLKR, 8), lambda i: (i, 0)),
        _full((1, D)),
        _full((B, A)),
        _full((1, D)), _full((1, D)),
        _full((A, 8 * A)), _full((1, 8 * A)),
        _full((1, 8 * A)), _full((1, 8 * A)),
        _full((8 * A, 8 * A)), _full((1, 8 * A)),
        _full((8 * A, D)), _full((1, D)),
        _full((D, FC)), _full((1, FC)),
        _full((1, FC)), _full((1, FC)),
        _full((FC, 1)), _full((1, 1)),
    ],
    out_specs=pl.BlockSpec((B, 1), lambda i: (0, 0)),
    out_shape=jax.ShapeDtypeStruct((B, 1), jnp.float32),
    scratch_shapes=[pltpu.VMEM((1, D), jnp.float32)],
)


def kernel(x, action, edge_index, W0, b0, W1, b1, W2, b2, W3, b3,
           ln2_g, ln2_b, afc1_W, afc1_b, ln5_g, ln5_b, afc2_W, afc2_b,
           afc3_W, afc3_b, fc1_W, fc1_b, ln4_g, ln4_b, fc2_W, fc2_b):
    src = edge_index[0]
    dst = edge_index[1]
    pad = EPAD - E
    src3 = jnp.concatenate(
        [src, jnp.zeros((pad,), jnp.int32)]).reshape(NW * NBLK, EBLK)
    dst3 = jnp.concatenate(
        [dst, jnp.full((pad,), N, jnp.int32)]).reshape(NW * NBLK, EBLK)

    zeros128 = jnp.zeros((ZBUF, DC), jnp.float32)
    ones128 = jnp.ones((EBLK, DC), jnp.float32)

    r1 = lambda a: a.reshape(1, -1)
    r3 = lambda outs: [a.reshape(NC, N, DC) for a in outs]

    deg = _sc_deg(dst3, zeros128, ones128).reshape(NC, NACC, DC)[:, :N]
    *hp, dinv = _tc_pre(x, W0, deg)

    agg = r3(_sc_agg(*hp, src3, dst3, zeros128))
    hp = _tc_mid_l0(*agg, *hp, dinv, r1(b0), W1)

    agg = r3(_sc_agg(*hp, src3, dst3, zeros128))
    hp = _tc_mid_act(*agg, *hp, dinv, r1(b1), W2)

    agg = r3(_sc_agg(*hp, src3, dst3, zeros128))
    hp = _tc_mid_act(*agg, *hp, dinv, r1(b2), W3)

    agg = r3(_sc_agg(*hp, src3, dst3, zeros128))
    out = _tc_final(*agg, *hp, dinv, r1(b3), action,
                    r1(ln2_g), r1(ln2_b), afc1_W, r1(afc1_b),
                    r1(ln5_g), r1(ln5_b), afc2_W, r1(afc2_b),
                    afc3_W, r1(afc3_b), fc1_W, r1(fc1_b),
                    r1(ln4_g), r1(ln4_b), fc2_W, r1(fc2_b))
    return out[:, 0]
